# Initial kernel scaffold; baseline (speedup 1.0000x reference)
import functools

import jax
import jax.numpy as jnp
from jax import lax
from jax.experimental import pallas as pl
from jax.experimental.pallas import tpu as pltpu
from jax.experimental.pallas import tpu_sc as plsc

_S = 200
_B = 16384
_D = 64
_N = _B * _S

_ROWS_PER_BLK = 32

_NC = 2
_NS = 16
_NW = _NC * _NS
_WIN = 128
_NWIN = _N // _WIN
_WPW = _NWIN // _NW
_CK = 100
_NCHUNK = _WPW // _CK


def _rank_body(ts_ref, out_ref):
    blk = pl.program_id(0)
    u = lax.bitcast_convert_type(ts_ref[...], jnp.int32)
    key = jnp.where(u < 0, jnp.bitwise_xor(jnp.bitwise_not(u), jnp.int32(-2147483648)), u)
    a = key[:, :, None]
    b = key[:, None, :]
    jlt = (lax.broadcasted_iota(jnp.int32, (_S, _S), 0)
           < lax.broadcasted_iota(jnp.int32, (_S, _S), 1))[None]
    before = (a < b) | ((a == b) & jlt)
    rank = jnp.sum(before.astype(jnp.int32), axis=1)
    row = blk * _ROWS_PER_BLK + lax.broadcasted_iota(
        jnp.int32, (_ROWS_PER_BLK, _S), 0)
    out_ref[...] = row * _S + rank


def _rank_tc(ts):
    return pl.pallas_call(
        _rank_body,
        grid=(_B // _ROWS_PER_BLK,),
        in_specs=[pl.BlockSpec((_ROWS_PER_BLK, _S), lambda i: (i, 0))],
        out_specs=pl.BlockSpec((_ROWS_PER_BLK, _S), lambda i: (i, 0)),
        out_shape=jax.ShapeDtypeStruct((_B, _S), jnp.int32),
    )(ts)


def _sc_scatter(table2, idx3):
    mesh = plsc.VectorSubcoreMesh(core_axis_name="c", subcore_axis_name="s")

    @functools.partial(
        pl.kernel,
        out_type=jax.ShapeDtypeStruct((_N, _D), jnp.float32),
        mesh=mesh,
        scratch_types=[
            pltpu.VMEM((2 * _S, _D), jnp.float32),
            pltpu.VMEM((_CK, _WIN), jnp.int32),
            pltpu.VMEM((_CK, _WIN), jnp.int32),
            pltpu.SemaphoreType.DMA,
            pltpu.SemaphoreType.DMA,
        ],
    )
    def k(table_hbm, idx_hbm, out_hbm, tab_v, ib0, ib1, sem0, sem1):
        wid = lax.axis_index("s") * _NC + lax.axis_index("c")
        pltpu.sync_copy(table_hbm, tab_v)
        my_idx = idx_hbm.at[wid]
        bufs = (ib0, ib1)
        sems = (sem0, sem1)
        pending = {0: pltpu.async_copy(my_idx.at[pl.ds(0, _CK)], ib0, sem0)}
        for c in range(_NCHUNK):
            pending.pop(c % 2).wait()
            if c + 1 < _NCHUNK:
                pending[(c + 1) % 2] = pltpu.async_copy(
                    my_idx.at[pl.ds((c + 1) * _CK, _CK)],
                    bufs[(c + 1) % 2], sems[(c + 1) % 2])
            buf = bufs[c % 2]
            base_win = wid * _WPW + c * _CK

            @pl.loop(0, _CK)
            def _(j2):
                start = lax.rem((base_win + j2) * _WIN, _S)
                pltpu.sync_copy(tab_v.at[pl.ds(start, _WIN)],
                                out_hbm.at[buf.at[j2]])

    return k(table2, idx3)


def kernel(ts, pos_emb_table):
    ts = ts.astype(jnp.float32)
    table = pos_emb_table.astype(jnp.float32)
    nonzero = jnp.any(ts != 0.0)
    eff = jnp.where(nonzero, table[:_S], jnp.broadcast_to(table[_S:_S + 1], (_S, _D)))
    table2 = jnp.concatenate([eff, eff], axis=0)

    dest = _rank_tc(ts)
    idx3 = dest.reshape(_NW, _WPW, _WIN)
    out = _sc_scatter(table2, idx3)
    return out.reshape(_B, _S, _D)


# speedup vs baseline: 4.4076x; 4.4076x over previous
import functools

import jax
import jax.numpy as jnp
from jax import lax
from jax.experimental import pallas as pl
from jax.experimental.pallas import tpu as pltpu
from jax.experimental.pallas import tpu_sc as plsc

_S = 200
_B = 16384
_D = 64
_N = _B * _S

_ROWS_PER_BLK = 32

_NC = 2
_NS = 16
_NW = _NC * _NS
_WIN = 128
_NWIN = _N // _WIN
_WPW = _NWIN // _NW
_CK = 80
_NCHUNK = _WPW // _CK


def _rank_body(ts_ref, out_ref):
    blk = pl.program_id(0)
    u = lax.bitcast_convert_type(ts_ref[...], jnp.int32)
    u = jnp.where(u == jnp.int32(-2147483648), jnp.int32(0), u)
    key = jnp.where(u < 0, jnp.bitwise_xor(jnp.bitwise_not(u), jnp.int32(-2147483648)), u)
    a = key[:, :, None]
    b = key[:, None, :]
    jlt = (lax.broadcasted_iota(jnp.int32, (_S, _S), 0)
           < lax.broadcasted_iota(jnp.int32, (_S, _S), 1))[None]
    before = (a < b) | ((a == b) & jlt)
    rank = jnp.sum(before.astype(jnp.int32), axis=1)
    row = blk * _ROWS_PER_BLK + lax.broadcasted_iota(
        jnp.int32, (_ROWS_PER_BLK, _S), 0)
    out_ref[...] = row * _S + rank


def _rank_tc(ts):
    return pl.pallas_call(
        _rank_body,
        grid=(_B // _ROWS_PER_BLK,),
        in_specs=[pl.BlockSpec((_ROWS_PER_BLK, _S), lambda i: (i, 0))],
        out_specs=pl.BlockSpec((_ROWS_PER_BLK, _S), lambda i: (i, 0)),
        out_shape=jax.ShapeDtypeStruct((_B, _S), jnp.int32),
    )(ts)


def _sc_scatter(table2, idx3):
    mesh = plsc.VectorSubcoreMesh(core_axis_name="c", subcore_axis_name="s")

    @functools.partial(
        pl.kernel,
        out_type=jax.ShapeDtypeStruct((_N, _D), jnp.float32),
        mesh=mesh,
        scratch_types=[
            pltpu.VMEM((2 * _S, _D), jnp.float32),
            pltpu.VMEM((_CK, _WIN), jnp.int32),
            pltpu.VMEM((_CK, _WIN), jnp.int32),
            pltpu.SemaphoreType.DMA,
            pltpu.SemaphoreType.DMA,
        ],
        compiler_params=pltpu.CompilerParams(use_tc_tiling_on_sc=False),
    )
    def k(table_hbm, idx_hbm, out_hbm, tab_v, ib0, ib1, sem0, sem1):
        wid = lax.axis_index("s") * _NC + lax.axis_index("c")
        pltpu.sync_copy(table_hbm, tab_v)
        my_idx = idx_hbm.at[wid]
        bufs = (ib0, ib1)
        sems = (sem0, sem1)
        pending = {0: pltpu.async_copy(my_idx.at[pl.ds(0, _CK)], ib0, sem0)}
        for c in range(_NCHUNK):
            pending.pop(c % 2).wait()
            if c + 1 < _NCHUNK:
                pending[(c + 1) % 2] = pltpu.async_copy(
                    my_idx.at[pl.ds((c + 1) * _CK, _CK)],
                    bufs[(c + 1) % 2], sems[(c + 1) % 2])
            buf = bufs[c % 2]
            base_win = wid * _WPW + c * _CK

            @pl.loop(0, _CK)
            def _(j2):
                start = lax.rem((base_win + j2) * _WIN, _S)
                pltpu.sync_copy(tab_v.at[pl.ds(start, _WIN)],
                                out_hbm.at[buf.at[j2]])

    return k(table2, idx3)


def kernel(ts, pos_emb_table):
    ts = ts.astype(jnp.float32)
    table = pos_emb_table.astype(jnp.float32)
    nonzero = jnp.any(ts != 0.0)
    eff = jnp.where(nonzero, table[:_S], jnp.broadcast_to(table[_S:_S + 1], (_S, _D)))
    table2 = jnp.concatenate([eff, eff], axis=0)

    dest = _rank_tc(ts)
    idx3 = dest.reshape(_NW, _WPW, _WIN)
    out = _sc_scatter(table2, idx3)
    return out.reshape(_B, _S, _D)


# speedup vs baseline: 6.4280x; 1.4584x over previous
import functools

import jax
import jax.numpy as jnp
from jax import lax
from jax.experimental import pallas as pl
from jax.experimental.pallas import tpu as pltpu
from jax.experimental.pallas import tpu_sc as plsc

_S = 200
_B = 16384
_D = 64
_N = _B * _S

_ROWS_PER_BLK = 32

_NC = 2
_NS = 16
_NW = _NC * _NS
_WIN = 128
_NWIN = _N // _WIN
_WPW = _NWIN // _NW
_CK = 80
_NCHUNK = _WPW // _CK


def _rank_body(ts_ref, out_ref):
    blk = pl.program_id(0)
    u = lax.bitcast_convert_type(ts_ref[...], jnp.int32)
    u = jnp.where(u == jnp.int32(-2147483648), jnp.int32(0), u)
    key = jnp.where(u < 0, jnp.bitwise_xor(jnp.bitwise_not(u), jnp.int32(-2147483648)), u)
    a = key[:, :, None]
    b = key[:, None, :]
    jlt = (lax.broadcasted_iota(jnp.int32, (_S, _S), 0)
           < lax.broadcasted_iota(jnp.int32, (_S, _S), 1))[None]
    before = (a < b) | ((a == b) & jlt)
    rank = jnp.sum(before.astype(jnp.int32), axis=1)
    row = blk * _ROWS_PER_BLK + lax.broadcasted_iota(
        jnp.int32, (_ROWS_PER_BLK, _S), 0)
    out_ref[...] = row * _S + rank


def _rank_tc(ts):
    return pl.pallas_call(
        _rank_body,
        grid=(_B // _ROWS_PER_BLK,),
        in_specs=[pl.BlockSpec((_ROWS_PER_BLK, _S), lambda i: (i, 0))],
        out_specs=pl.BlockSpec((_ROWS_PER_BLK, _S), lambda i: (i, 0)),
        out_shape=jax.ShapeDtypeStruct((_B, _S), jnp.int32),
    )(ts)


def _sc_scatter(table2, idx3):
    mesh = plsc.VectorSubcoreMesh(core_axis_name="c", subcore_axis_name="s")

    @functools.partial(
        pl.kernel,
        out_type=jax.ShapeDtypeStruct((_N, 128), jnp.float32),
        mesh=mesh,
        scratch_types=[
            pltpu.VMEM((2 * _S, 128), jnp.float32),
            pltpu.VMEM((_CK, _WIN), jnp.int32),
            pltpu.VMEM((_CK, _WIN), jnp.int32),
            pltpu.SemaphoreType.DMA,
            pltpu.SemaphoreType.DMA,
        ],
        compiler_params=pltpu.CompilerParams(use_tc_tiling_on_sc=True),
    )
    def k(table_hbm, idx_hbm, out_hbm, tab_v, ib0, ib1, sem0, sem1):
        wid = lax.axis_index("s") * _NC + lax.axis_index("c")
        pltpu.sync_copy(table_hbm, tab_v)
        my_idx = idx_hbm.at[wid]
        bufs = (ib0, ib1)
        sems = (sem0, sem1)
        pending = {0: pltpu.async_copy(my_idx.at[pl.ds(0, _CK)], ib0, sem0)}
        for c in range(_NCHUNK):
            pending.pop(c % 2).wait()
            if c + 1 < _NCHUNK:
                pending[(c + 1) % 2] = pltpu.async_copy(
                    my_idx.at[pl.ds((c + 1) * _CK, _CK)],
                    bufs[(c + 1) % 2], sems[(c + 1) % 2])
            buf = bufs[c % 2]
            base_win = wid * _WPW + c * _CK

            @pl.loop(0, _CK)
            def _(j2):
                start = lax.rem((base_win + j2) * _WIN, _S)
                pltpu.sync_copy(tab_v.at[pl.ds(start, _WIN)],
                                out_hbm.at[buf.at[j2]])

    return k(table2, idx3)


def kernel(ts, pos_emb_table):
    ts = ts.astype(jnp.float32)
    table = pos_emb_table.astype(jnp.float32)
    nonzero = jnp.any(ts != 0.0)
    eff = jnp.where(nonzero, table[:_S], jnp.broadcast_to(table[_S:_S + 1], (_S, _D)))
    table2 = jnp.concatenate([eff, eff], axis=0)
    table2 = jnp.pad(table2, ((0, 0), (0, 128 - _D)))

    dest = _rank_tc(ts)
    idx3 = dest.reshape(_NW, _WPW, _WIN)
    out2 = _sc_scatter(table2, idx3)
    return out2[:, :_D].reshape(_B, _S, _D)


# speedup vs baseline: 6.4404x; 1.0019x over previous
import functools

import jax
import jax.numpy as jnp
from jax import lax
from jax.experimental import pallas as pl
from jax.experimental.pallas import tpu as pltpu
from jax.experimental.pallas import tpu_sc as plsc

_S = 200
_B = 16384
_D = 64
_N = _B * _S

_ROWS_PER_BLK = 32

_NC = 2
_NS = 16
_NW = _NC * _NS
_WIN = 128
_NWIN = _N // _WIN
_WPW = _NWIN // _NW
_CK = 80
_NCHUNK = _WPW // _CK


def _rank_body(ts_ref, out_ref):
    blk = pl.program_id(0)
    u = lax.bitcast_convert_type(ts_ref[...], jnp.int32)
    u = jnp.where(u == jnp.int32(-2147483648), jnp.int32(0), u)
    key = jnp.where(u < 0, jnp.bitwise_xor(jnp.bitwise_not(u), jnp.int32(-2147483648)), u)
    a = key[:, :, None]
    b = key[:, None, :]
    jlt = (lax.broadcasted_iota(jnp.int32, (_S, _S), 0)
           < lax.broadcasted_iota(jnp.int32, (_S, _S), 1))[None]
    before = (a < b) | ((a == b) & jlt)
    rank = jnp.sum(before.astype(jnp.int32), axis=1)
    row = blk * _ROWS_PER_BLK + lax.broadcasted_iota(
        jnp.int32, (_ROWS_PER_BLK, _S), 0)
    out_ref[...] = row * _S + rank


def _rank_tc(ts):
    return pl.pallas_call(
        _rank_body,
        grid=(_B // _ROWS_PER_BLK,),
        in_specs=[pl.BlockSpec((_ROWS_PER_BLK, _S), lambda i: (i, 0))],
        out_specs=pl.BlockSpec((_ROWS_PER_BLK, _S), lambda i: (i, 0)),
        out_shape=jax.ShapeDtypeStruct((_B, _S), jnp.int32),
        compiler_params=pltpu.CompilerParams(
            dimension_semantics=("parallel",)),
    )(ts)


def _sc_scatter(table2, idx3):
    mesh = plsc.VectorSubcoreMesh(core_axis_name="c", subcore_axis_name="s")

    @functools.partial(
        pl.kernel,
        out_type=jax.ShapeDtypeStruct((_N, 128), jnp.float32),
        mesh=mesh,
        scratch_types=[
            pltpu.VMEM((2 * _S, 128), jnp.float32),
            pltpu.VMEM((_CK, _WIN), jnp.int32),
            pltpu.VMEM((_CK, _WIN), jnp.int32),
            pltpu.SemaphoreType.DMA,
            pltpu.SemaphoreType.DMA,
        ],
        compiler_params=pltpu.CompilerParams(use_tc_tiling_on_sc=True),
    )
    def k(table_hbm, idx_hbm, out_hbm, tab_v, ib0, ib1, sem0, sem1):
        wid = lax.axis_index("s") * _NC + lax.axis_index("c")
        pltpu.sync_copy(table_hbm, tab_v)
        my_idx = idx_hbm.at[wid]
        bufs = (ib0, ib1)
        sems = (sem0, sem1)
        pending = {0: pltpu.async_copy(my_idx.at[pl.ds(0, _CK)], ib0, sem0)}
        for c in range(_NCHUNK):
            pending.pop(c % 2).wait()
            if c + 1 < _NCHUNK:
                pending[(c + 1) % 2] = pltpu.async_copy(
                    my_idx.at[pl.ds((c + 1) * _CK, _CK)],
                    bufs[(c + 1) % 2], sems[(c + 1) % 2])
            buf = bufs[c % 2]
            base_win = wid * _WPW + c * _CK

            @pl.loop(0, _CK)
            def _(j2):
                start = lax.rem((base_win + j2) * _WIN, _S)
                pltpu.sync_copy(tab_v.at[pl.ds(start, _WIN)],
                                out_hbm.at[buf.at[j2]])

    return k(table2, idx3)


def kernel(ts, pos_emb_table):
    ts = ts.astype(jnp.float32)
    table = pos_emb_table.astype(jnp.float32)
    nonzero = jnp.any(ts != 0.0)
    eff = jnp.where(nonzero, table[:_S], jnp.broadcast_to(table[_S:_S + 1], (_S, _D)))
    table2 = jnp.concatenate([eff, eff], axis=0)
    table2 = jnp.pad(table2, ((0, 0), (0, 128 - _D)))

    dest = _rank_tc(ts)
    idx3 = dest.reshape(_NW, _WPW, _WIN)
    out2 = _sc_scatter(table2, idx3)
    return out2[:, :_D].reshape(_B, _S, _D)


# speedup vs baseline: 7.7870x; 1.2091x over previous
import functools

import jax
import jax.numpy as jnp
from jax import lax
from jax.experimental import pallas as pl
from jax.experimental.pallas import tpu as pltpu
from jax.experimental.pallas import tpu_sc as plsc

_S = 200
_B = 16384
_D = 64
_N = _B * _S

_ROWS_PER_BLK = 32

_NC = 2
_NS = 16
_NW = _NC * _NS
_WIN = 128

_NCHUNK = 4
_BC = _B // _NCHUNK
_NWINC = _BC * _S // _WIN
_WPW = _NWINC // _NW
_CK = 40
_NIC = _WPW // _CK


def _rank_body(ts_ref, out_ref):
    blk = pl.program_id(0)
    u = lax.bitcast_convert_type(ts_ref[...], jnp.int32)
    u = jnp.where(u == jnp.int32(-2147483648), jnp.int32(0), u)
    key = jnp.where(u < 0, jnp.bitwise_xor(jnp.bitwise_not(u), jnp.int32(-2147483648)), u)
    a = key[:, :, None]
    b = key[:, None, :]
    jlt = (lax.broadcasted_iota(jnp.int32, (_S, _S), 0)
           < lax.broadcasted_iota(jnp.int32, (_S, _S), 1))[None]
    before = (a < b) | ((a == b) & jlt)
    rank = jnp.sum(before.astype(jnp.int32), axis=1)
    row = blk * _ROWS_PER_BLK + lax.broadcasted_iota(
        jnp.int32, (_ROWS_PER_BLK, _S), 0)
    out_ref[...] = row * _S + rank


def _rank_tc(ts_chunk, chunk):
    def body(ts_ref, out_ref):
        blk = pl.program_id(0)
        u = lax.bitcast_convert_type(ts_ref[...], jnp.int32)
        u = jnp.where(u == jnp.int32(-2147483648), jnp.int32(0), u)
        key = jnp.where(u < 0, jnp.bitwise_xor(jnp.bitwise_not(u), jnp.int32(-2147483648)), u)
        a = key[:, :, None]
        b = key[:, None, :]
        jlt = (lax.broadcasted_iota(jnp.int32, (_S, _S), 0)
               < lax.broadcasted_iota(jnp.int32, (_S, _S), 1))[None]
        before = (a < b) | ((a == b) & jlt)
        rank = jnp.sum(before.astype(jnp.int32), axis=1)
        row = (chunk * _BC + blk * _ROWS_PER_BLK
               + lax.broadcasted_iota(jnp.int32, (_ROWS_PER_BLK, _S), 0))
        out_ref[...] = row * _S + rank

    return pl.pallas_call(
        body,
        grid=(_BC // _ROWS_PER_BLK,),
        in_specs=[pl.BlockSpec((_ROWS_PER_BLK, _S), lambda i: (i, 0))],
        out_specs=pl.BlockSpec((_ROWS_PER_BLK, _S), lambda i: (i, 0)),
        out_shape=jax.ShapeDtypeStruct((_BC, _S), jnp.int32),
        compiler_params=pltpu.CompilerParams(
            dimension_semantics=("parallel",)),
    )(ts_chunk)


def _make_scatter(first):
    mesh = plsc.VectorSubcoreMesh(core_axis_name="c", subcore_axis_name="s")
    out_type = (jax.ShapeDtypeStruct((_N, 128), jnp.float32) if first else ())

    @functools.partial(
        pl.kernel,
        out_type=out_type,
        mesh=mesh,
        scratch_types=[
            pltpu.VMEM((2 * _S, 128), jnp.float32),
            pltpu.VMEM((_CK, _WIN), jnp.int32),
            pltpu.VMEM((_CK, _WIN), jnp.int32),
            pltpu.SemaphoreType.DMA,
            pltpu.SemaphoreType.DMA,
        ],
        compiler_params=pltpu.CompilerParams(use_tc_tiling_on_sc=True),
    )
    def k(table_hbm, idx_hbm, out_hbm, tab_v, ib0, ib1, sem0, sem1):
        wid = lax.axis_index("s") * _NC + lax.axis_index("c")
        pltpu.sync_copy(table_hbm, tab_v)
        my_idx = idx_hbm.at[wid]
        bufs = (ib0, ib1)
        sems = (sem0, sem1)
        pending = {0: pltpu.async_copy(my_idx.at[pl.ds(0, _CK)], ib0, sem0)}
        for c in range(_NIC):
            pending.pop(c % 2).wait()
            if c + 1 < _NIC:
                pending[(c + 1) % 2] = pltpu.async_copy(
                    my_idx.at[pl.ds((c + 1) * _CK, _CK)],
                    bufs[(c + 1) % 2], sems[(c + 1) % 2])
            buf = bufs[c % 2]
            base_win = wid * _WPW + c * _CK

            @pl.loop(0, _CK)
            def _(j2):
                start = lax.rem((base_win + j2) * _WIN, _S)
                pltpu.sync_copy(tab_v.at[pl.ds(start, _WIN)],
                                out_hbm.at[buf.at[j2]])

    return k


_scatter_first = _make_scatter(True)
_scatter_next = _make_scatter(False)


def kernel(ts, pos_emb_table):
    ts = ts.astype(jnp.float32)
    table = pos_emb_table.astype(jnp.float32)
    nonzero = jnp.any(ts != 0.0)
    eff = jnp.where(nonzero, table[:_S], jnp.broadcast_to(table[_S:_S + 1], (_S, _D)))
    table2 = jnp.concatenate([eff, eff], axis=0)
    table2 = jnp.pad(table2, ((0, 0), (0, 128 - _D)))

    dest0 = _rank_tc(ts[0:_BC], 0)
    out2 = _scatter_first(table2, dest0.reshape(_NW, _WPW, _WIN))
    out_ref = jax.new_ref(out2)
    for c in range(1, _NCHUNK):
        dest = _rank_tc(ts[c * _BC:(c + 1) * _BC], c)
        _scatter_next(table2, dest.reshape(_NW, _WPW, _WIN), out_ref)
    out2 = out_ref[...]
    return out2[:, :_D].reshape(_B, _S, _D)


# speedup vs baseline: 7.8729x; 1.0110x over previous
import functools

import jax
import jax.numpy as jnp
from jax import lax
from jax.experimental import pallas as pl
from jax.experimental.pallas import tpu as pltpu
from jax.experimental.pallas import tpu_sc as plsc

_S = 200
_B = 16384
_D = 64
_N = _B * _S

_ROWS_PER_BLK = 64

_NC = 2
_NS = 16
_NW = _NC * _NS
_WIN = 128

_NCHUNK = 4
_BC = _B // _NCHUNK
_NWINC = _BC * _S // _WIN
_WPW = _NWINC // _NW
_CK = 40
_NIC = _WPW // _CK


def _rank_body(ts_ref, out_ref):
    blk = pl.program_id(0)
    u = lax.bitcast_convert_type(ts_ref[...], jnp.int32)
    u = jnp.where(u == jnp.int32(-2147483648), jnp.int32(0), u)
    key = jnp.where(u < 0, jnp.bitwise_xor(jnp.bitwise_not(u), jnp.int32(-2147483648)), u)
    a = key[:, :, None]
    b = key[:, None, :]
    jlt = (lax.broadcasted_iota(jnp.int32, (_S, _S), 0)
           < lax.broadcasted_iota(jnp.int32, (_S, _S), 1))[None]
    before = (a < b) | ((a == b) & jlt)
    rank = jnp.sum(before.astype(jnp.int32), axis=1)
    row = blk * _ROWS_PER_BLK + lax.broadcasted_iota(
        jnp.int32, (_ROWS_PER_BLK, _S), 0)
    out_ref[...] = row * _S + rank


def _rank_tc(ts_chunk, chunk):
    def body(ts_ref, out_ref):
        blk = pl.program_id(0)
        u = lax.bitcast_convert_type(ts_ref[...], jnp.int32)
        u = jnp.where(u == jnp.int32(-2147483648), jnp.int32(0), u)
        key = jnp.where(u < 0, jnp.bitwise_xor(jnp.bitwise_not(u), jnp.int32(-2147483648)), u)
        a = key[:, :, None]
        b = key[:, None, :]
        jlt = (lax.broadcasted_iota(jnp.int32, (_S, _S), 0)
               < lax.broadcasted_iota(jnp.int32, (_S, _S), 1))[None]
        before = (a < b) | ((a == b) & jlt)
        rank = jnp.sum(before.astype(jnp.int32), axis=1)
        row = (chunk * _BC + blk * _ROWS_PER_BLK
               + lax.broadcasted_iota(jnp.int32, (_ROWS_PER_BLK, _S), 0))
        out_ref[...] = row * _S + rank

    return pl.pallas_call(
        body,
        grid=(_BC // _ROWS_PER_BLK,),
        in_specs=[pl.BlockSpec((_ROWS_PER_BLK, _S), lambda i: (i, 0))],
        out_specs=pl.BlockSpec((_ROWS_PER_BLK, _S), lambda i: (i, 0)),
        out_shape=jax.ShapeDtypeStruct((_BC, _S), jnp.int32),
        compiler_params=pltpu.CompilerParams(
            dimension_semantics=("parallel",)),
    )(ts_chunk)


def _make_scatter(first):
    mesh = plsc.VectorSubcoreMesh(core_axis_name="c", subcore_axis_name="s")
    out_type = (jax.ShapeDtypeStruct((_N, 128), jnp.float32) if first else ())

    @functools.partial(
        pl.kernel,
        out_type=out_type,
        mesh=mesh,
        scratch_types=[
            pltpu.VMEM((2 * _S, 128), jnp.float32),
            pltpu.VMEM((_CK, _WIN), jnp.int32),
            pltpu.VMEM((_CK, _WIN), jnp.int32),
            pltpu.SemaphoreType.DMA,
            pltpu.SemaphoreType.DMA,
        ],
        compiler_params=pltpu.CompilerParams(use_tc_tiling_on_sc=True),
    )
    def k(table_hbm, idx_hbm, out_hbm, tab_v, ib0, ib1, sem0, sem1):
        wid = lax.axis_index("s") * _NC + lax.axis_index("c")
        pltpu.sync_copy(table_hbm, tab_v)
        my_idx = idx_hbm.at[wid]
        bufs = (ib0, ib1)
        sems = (sem0, sem1)
        pending = {0: pltpu.async_copy(my_idx.at[pl.ds(0, _CK)], ib0, sem0)}
        for c in range(_NIC):
            pending.pop(c % 2).wait()
            if c + 1 < _NIC:
                pending[(c + 1) % 2] = pltpu.async_copy(
                    my_idx.at[pl.ds((c + 1) * _CK, _CK)],
                    bufs[(c + 1) % 2], sems[(c + 1) % 2])
            buf = bufs[c % 2]
            base_win = wid * _WPW + c * _CK

            @pl.loop(0, _CK)
            def _(j2):
                start = lax.rem((base_win + j2) * _WIN, _S)
                pltpu.sync_copy(tab_v.at[pl.ds(start, _WIN)],
                                out_hbm.at[buf.at[j2]])

    return k


_scatter_first = _make_scatter(True)
_scatter_next = _make_scatter(False)


def kernel(ts, pos_emb_table):
    ts = ts.astype(jnp.float32)
    table = pos_emb_table.astype(jnp.float32)
    nonzero = jnp.any(ts != 0.0)
    eff = jnp.where(nonzero, table[:_S], jnp.broadcast_to(table[_S:_S + 1], (_S, _D)))
    table2 = jnp.concatenate([eff, eff], axis=0)
    table2 = jnp.pad(table2, ((0, 0), (0, 128 - _D)))

    dest0 = _rank_tc(ts[0:_BC], 0)
    out2 = _scatter_first(table2, dest0.reshape(_NW, _WPW, _WIN))
    out_ref = jax.new_ref(out2)
    for c in range(1, _NCHUNK):
        dest = _rank_tc(ts[c * _BC:(c + 1) * _BC], c)
        _scatter_next(table2, dest.reshape(_NW, _WPW, _WIN), out_ref)
    out2 = out_ref[...]
    return out2[:, :_D].reshape(_B, _S, _D)


# speedup vs baseline: 9.3159x; 1.1833x over previous
import functools

import jax
import jax.numpy as jnp
from jax import lax
from jax.experimental import pallas as pl
from jax.experimental.pallas import tpu as pltpu
from jax.experimental.pallas import tpu_sc as plsc

_S = 200
_B = 16384
_D = 64
_N = _B * _S

_BL = 128
_JC = 8

_NC = 2
_NS = 16
_NW = _NC * _NS
_WIN = 128

_NCHUNK = 4
_BC = _B // _NCHUNK
_WPK = _BC // _WIN
_KPW = (_S + _NW - 1) // _NW


def _rank_tc(tsT_chunk, chunk):
    def body(ts_ref, out_ref):
        blk = pl.program_id(0)
        u = lax.bitcast_convert_type(ts_ref[...], jnp.int32)
        u = jnp.where(u == jnp.int32(-2147483648), jnp.int32(0), u)
        key = jnp.where(
            u < 0, jnp.bitwise_xor(jnp.bitwise_not(u), jnp.int32(-2147483648)), u)
        b3 = key[None]
        kio = lax.broadcasted_iota(jnp.int32, (_JC, _S, _BL), 1)
        jio = lax.broadcasted_iota(jnp.int32, (_JC, _S, _BL), 0)
        acc = jnp.zeros((_S, _BL), jnp.int32)
        for jb in range(0, _S, _JC):
            a3 = key[jb:jb + _JC][:, None, :]
            jlt = (jio + jb) < kio
            cmp = (a3 < b3) | ((a3 == b3) & jlt)
            acc = acc + jnp.sum(cmp.astype(jnp.int32), axis=0)
        bvec = (chunk * _BC + blk * _BL
                + lax.broadcasted_iota(jnp.int32, (_S, _BL), 1))
        out_ref[...] = bvec * _S + acc

    return pl.pallas_call(
        body,
        grid=(_BC // _BL,),
        in_specs=[pl.BlockSpec((_S, _BL), lambda i: (0, i))],
        out_specs=pl.BlockSpec((_S, _BL), lambda i: (0, i)),
        out_shape=jax.ShapeDtypeStruct((_S, _BC), jnp.int32),
        compiler_params=pltpu.CompilerParams(
            dimension_semantics=("parallel",)),
    )(tsT_chunk)


def _make_scatter(first):
    mesh = plsc.VectorSubcoreMesh(core_axis_name="c", subcore_axis_name="s")
    out_type = (jax.ShapeDtypeStruct((_N, 128), jnp.float32) if first else ())

    @functools.partial(
        pl.kernel,
        out_type=out_type,
        mesh=mesh,
        scratch_types=[
            pltpu.VMEM((_WIN, 128), jnp.float32),
            pltpu.VMEM((_WPK, _WIN), jnp.int32),
            pltpu.SemaphoreType.DMA,
            pltpu.SemaphoreType.DMA,
        ],
        compiler_params=pltpu.CompilerParams(use_tc_tiling_on_sc=True),
    )
    def k(repl_hbm, idx_hbm, out_hbm, repl_v, ib, sem0, sem1):
        wid = lax.axis_index("s") * _NC + lax.axis_index("c")
        for t in range(_KPW):
            krow = wid + t * _NW

            @pl.when(krow < _S)
            def _():
                base = pl.multiple_of(krow * _WPK, 8)
                cp_i = pltpu.async_copy(idx_hbm.at[pl.ds(base, _WPK)], ib, sem0)
                cp_r = pltpu.async_copy(repl_hbm.at[krow], repl_v, sem1)
                cp_i.wait()
                cp_r.wait()

                @pl.loop(0, _WPK)
                def _(m):
                    pltpu.sync_copy(repl_v, out_hbm.at[ib.at[m]])

    return k


_scatter_first = _make_scatter(True)
_scatter_next = _make_scatter(False)


def kernel(ts, pos_emb_table):
    ts = ts.astype(jnp.float32)
    table = pos_emb_table.astype(jnp.float32)
    nonzero = jnp.any(ts != 0.0)
    eff = jnp.where(nonzero, table[:_S], jnp.broadcast_to(table[_S:_S + 1], (_S, _D)))
    eff = jnp.pad(eff, ((0, 0), (0, 128 - _D)))
    repl = jnp.broadcast_to(eff[:, None, :], (_S, _WIN, 128))

    tsT = ts.T
    dest0 = _rank_tc(tsT[:, 0:_BC], 0)
    out2 = _scatter_first(repl, dest0.reshape(_S * _WPK, _WIN))
    out_ref = jax.new_ref(out2)
    for c in range(1, _NCHUNK):
        dest = _rank_tc(tsT[:, c * _BC:(c + 1) * _BC], c)
        _scatter_next(repl, dest.reshape(_S * _WPK, _WIN), out_ref)
    out2 = out_ref[...]
    return out2[:, :_D].reshape(_B, _S, _D)
